# baseline (speedup 1.0000x reference)
import jax
import jax.numpy as jnp
from jax import lax
from jax.experimental import pallas as pl
from jax.experimental.pallas import tpu as pltpu
from jax.experimental.pallas import tpu_sc as plsc

NC = 2
NS = 16
NW = NC * NS
LANES = 16

C = 2000
VITERS = C // LANES

_MAGIC = 0x5F3759DF


def _rsqrt(x):
    xi = lax.bitcast_convert_type(x, jnp.int32)
    y = lax.bitcast_convert_type(
        jnp.int32(_MAGIC) - lax.shift_right_logical(xi, 1), jnp.float32)
    for _ in range(3):
        y = y * (1.5 - 0.5 * x * y * y)
    return y


def _make_beam_body(chunk_off, cpw):
  def _beam_body(*refs):
    (tdx, tdy, tth, tcx, tcz, na1, nb1, p_e, p_a, p_i, z1) = refs[:11]
    outs_o = refs[11:32]
    (p0x_o, p0y_o, p0z_o, p1x_o, p1y_o, p1z_o) = refs[32:38]
    scr = refs[38:]
    ins = [list(scr[s * 15:(s + 1) * 15]) for s in range(2)]
    outs = list(scr[30:49])
    accx, accy, accz = scr[49:52]
    sem_lin, sem_gat, sem_out, sem_sc = scr[52:56]

    core = lax.axis_index("c")
    sub = lax.axis_index("s")
    wid = sub * NC + core
    assert cpw >= 3

    tables = (tdx, tdy, tth, tcx, tcz)

    def lin_pairs(ck, s):
        sl = pl.ds((chunk_off + wid * cpw + ck) * C, C)
        bufs = ins[s]
        return [(na1.at[sl], bufs[0]), (nb1.at[sl], bufs[1]),
                (p_e.at[sl], bufs[2]), (p_a.at[sl], bufs[3]),
                (p_i.at[sl], bufs[4])]

    def gat_pairs(s):
        bufs = ins[s]
        prs = [(tab.at[bufs[0]], bufs[5 + t]) for t, tab in enumerate(tables)]
        prs += [(tab.at[bufs[1]], bufs[10 + t]) for t, tab in enumerate(tables)]
        return prs

    def out_pairs(ck, s):
        sl = pl.ds((wid * cpw + ck) * C, C)
        prs = [(buf, ref.at[sl]) for buf, ref in zip(outs, (
            outs_o[0], outs_o[1], outs_o[2], outs_o[3], outs_o[4], outs_o[5],
            outs_o[6], outs_o[7], outs_o[8], outs_o[9],
            outs_o[10], outs_o[11], outs_o[13], outs_o[14],
            outs_o[16], outs_o[17], outs_o[18], outs_o[19], outs_o[20]))]
        prs += [(ins[s][7], outs_o[12].at[sl]),
                (ins[s][12], outs_o[15].at[sl])]
        return prs

    def sc_pairs(s):
        ia, ib = ins[s][0], ins[s][1]
        return [(outs[6], accx.at[ia]), (outs[7], accy.at[ia]),
                (outs[2], accz.at[ia]), (outs[8], accx.at[ib]),
                (outs[9], accy.at[ib]), (outs[5], accz.at[ib])]

    def issue(pairs, sem, add=False):
        for src, dst in pairs:
            pltpu.async_copy(src, dst, sem, add=add)

    def drain(pairs, sem):
        for src, dst in pairs:
            pltpu.make_async_copy(src, dst, sem).wait()

    def compute(s):
        bufs = ins[s]
        (f0_v, f1_v, f2_v, f3_v, f4_v, f5_v, g0_v, g1_v, g3_v, g4_v,
         ua_v, wa_v, ub_v, wb_v, ul_v, phi_v, l0_v, cc_v, ss_v) = outs

        def vec(i, _):
            sl = pl.ds(i * LANES, LANES)
            d_ax = bufs[5][sl]
            d_ay = bufs[6][sl]
            ta = bufs[7][sl]
            cax = bufs[8][sl]
            caz = bufs[9][sl]
            d_bx = bufs[10][sl]
            d_by = bufs[11][sl]
            tb = bufs[12][sl]
            cbx = bufs[13][sl]
            cbz = bufs[14][sl]
            pe = bufs[2][sl]
            pa = bufs[3][sl]
            pi = bufs[4][sl]

            dx0 = cbx - cax
            dz0 = cbz - caz
            x = dx0 * dx0 + dz0 * dz0
            rl = _rsqrt(x)
            l0 = x * rl
            cv = dx0 * rl
            sv = dz0 * rl
            ea = pe * pa
            ei = pe * pi
            ua = cv * d_ax + sv * d_ay
            wa = cv * d_ay - sv * d_ax
            ub = cv * d_bx + sv * d_by
            wb = cv * d_by - sv * d_bx
            rl2 = rl * rl
            rl3 = rl2 * rl
            f0 = ea * rl * (ua - ub)
            wab = wa - wb
            eil3 = ei * rl3
            eil2 = ei * rl2
            eil = ei * rl
            f1 = 12.0 * eil3 * wab + 6.0 * eil2 * (ta + tb)
            f2 = 6.0 * eil2 * wab + eil * (4.0 * ta + 2.0 * tb)
            f5 = 6.0 * eil2 * wab + eil * (2.0 * ta + 4.0 * tb)
            f3 = -f0
            f4 = -f1
            g0 = cv * f0 - sv * f1
            g1 = sv * f0 + cv * f1
            g3 = -g0
            g4 = -g1

            f0_v[sl] = f0
            f1_v[sl] = f1
            f2_v[sl] = f2
            f3_v[sl] = f3
            f4_v[sl] = f4
            f5_v[sl] = f5
            g0_v[sl] = g0
            g1_v[sl] = g1
            g3_v[sl] = g3
            g4_v[sl] = g4
            ua_v[sl] = ua
            wa_v[sl] = wa
            ub_v[sl] = ub
            wb_v[sl] = wb
            ul_v[sl] = ub - ua
            phi_v[sl] = (wb - wa) * rl
            l0_v[sl] = l0
            cc_v[sl] = cv
            ss_v[sl] = sv
            return 0

        lax.fori_loop(0, VITERS, vec, 0)

    @pl.when(sub == 0)
    def _zero():
        pltpu.sync_copy(z1, accx)
        pltpu.sync_copy(z1, accy)
        pltpu.sync_copy(z1, accz)

    plsc.subcore_barrier()

    issue(lin_pairs(0, 0), sem_lin)
    drain(lin_pairs(0, 0), sem_lin)
    issue(gat_pairs(0), sem_gat)
    issue(lin_pairs(1, 1), sem_lin)
    drain(gat_pairs(0), sem_gat)
    compute(0)
    issue(out_pairs(0, 0), sem_out)
    issue(sc_pairs(0), sem_sc, add=True)
    drain(lin_pairs(1, 1), sem_lin)
    issue(gat_pairs(1), sem_gat)

    def body(k, _):
        j1 = 2 * k + 1
        j2 = 2 * k + 2
        drain(out_pairs(j1 - 1, 0), sem_out)
        drain(sc_pairs(0), sem_sc)
        issue(lin_pairs(j2, 0), sem_lin)
        drain(gat_pairs(1), sem_gat)
        compute(1)
        issue(out_pairs(j1, 1), sem_out)
        issue(sc_pairs(1), sem_sc, add=True)
        drain(lin_pairs(j2, 0), sem_lin)
        issue(gat_pairs(0), sem_gat)
        drain(out_pairs(j1, 1), sem_out)
        drain(sc_pairs(1), sem_sc)

        @pl.when(2 * k + 3 < cpw)
        def _pref():
            issue(lin_pairs(j2 + 1, 1), sem_lin)

        drain(gat_pairs(0), sem_gat)
        compute(0)
        issue(out_pairs(j2, 0), sem_out)
        issue(sc_pairs(0), sem_sc, add=True)

        @pl.when(2 * k + 3 < cpw)
        def _gat():
            drain(lin_pairs(j2 + 1, 1), sem_lin)
            issue(gat_pairs(1), sem_gat)

        return 0

    lax.fori_loop(0, (cpw - 1) // 2, body, 0)
    if cpw % 2 == 0:
        jt = cpw - 1
        drain(out_pairs(jt - 1, 0), sem_out)
        drain(sc_pairs(0), sem_sc)
        drain(gat_pairs(1), sem_gat)
        compute(1)
        issue(out_pairs(jt, 1), sem_out)
        issue(sc_pairs(1), sem_sc, add=True)
        drain(out_pairs(jt, 1), sem_out)
        drain(sc_pairs(1), sem_sc)
    else:
        drain(out_pairs(cpw - 1, 0), sem_out)
        drain(sc_pairs(0), sem_sc)

    plsc.subcore_barrier()

    @pl.when(jnp.logical_and(sub == 0, core == 0))
    def _out0():
        pltpu.sync_copy(accx, p0x_o)
        pltpu.sync_copy(accy, p0y_o)
        pltpu.sync_copy(accz, p0z_o)

    @pl.when(jnp.logical_and(sub == 0, core == 1))
    def _out1():
        pltpu.sync_copy(accx, p1x_o)
        pltpu.sync_copy(accy, p1y_o)
        pltpu.sync_copy(accz, p1z_o)

  return _beam_body


def kernel(pred_disp, coords, prop_E, prop_A, prop_I22, connectivity):
    n_nodes = pred_disp.shape[0]
    n_edges = connectivity.shape[0]
    total_chunks = n_edges // C
    assert n_edges % (NW * C) == 0

    f32 = jnp.float32
    tdx = pred_disp[:, 0].astype(f32)
    tdy = pred_disp[:, 1].astype(f32)
    tth = pred_disp[:, 2].astype(f32)
    tcx = coords[:, 0].astype(f32)
    tcz = coords[:, 2].astype(f32)
    na1 = connectivity[:, 0].astype(jnp.int32)
    nb1 = connectivity[:, 1].astype(jnp.int32)
    z1 = jnp.zeros((n_nodes,), f32)
    args = (tdx, tdy, tth, tcx, tcz, na1, nb1,
            prop_E.astype(f32), prop_A.astype(f32), prop_I22.astype(f32), z1)

    n1 = jax.ShapeDtypeStruct((n_nodes,), f32)
    set_bufs = ([pltpu.VMEM((C,), jnp.int32)] * 2 +
                [pltpu.VMEM((C,), f32)] * 3 +
                [pltpu.VMEM((C,), f32)] * 10)
    scratch = (
        set_bufs + set_bufs +
        [pltpu.VMEM((C,), f32)] * 19 +
        [pltpu.VMEM_SHARED((n_nodes,), f32)] * 3 +
        [pltpu.SemaphoreType.DMA] * 4
    )
    mesh = plsc.VectorSubcoreMesh(core_axis_name="c", subcore_axis_name="s",
                                  num_cores=NC, num_subcores=NS)

    def run_part(chunk_off, cpw):
        e1 = jax.ShapeDtypeStruct((cpw * NW * C,), f32)
        out_type = (e1,) * 21 + (n1,) * 6
        run = pl.kernel(_make_beam_body(chunk_off, cpw), out_type=out_type,
                        mesh=mesh, scratch_types=scratch,
                        compiler_params=pltpu.CompilerParams(
                            needs_layout_passes=False))
        return run(*args)

    cpw_a = (total_chunks // NW + 1) // 2
    cpw_b = total_chunks // NW - cpw_a
    res_a = run_part(0, cpw_a)
    res_b = run_part(cpw_a * NW, cpw_b)

    def cat(i):
        return jnp.concatenate([res_a[i], res_b[i]])

    def stk(idxs):
        ha = jnp.stack([res_a[i] for i in idxs], axis=1)
        hb = jnp.stack([res_b[i] for i in idxs], axis=1)
        return jnp.concatenate([ha, hb], axis=0)

    f_local = stk([0, 1, 2, 3, 4, 5])
    f_global = stk([6, 7, 2, 8, 9, 5])
    d_local = stk([10, 11, 12, 13, 14, 15])
    f_ga = f_global[:, 0:3]
    f_gb = f_global[:, 3:6]
    pa = res_a[21:]
    pb = res_b[21:]
    nodal_forces = jnp.stack([pa[0] + pa[3] + pb[0] + pb[3],
                              pa[1] + pa[4] + pb[1] + pb[4],
                              pa[2] + pa[5] + pb[2] + pb[5]], axis=1)
    return (cat(3), cat(2), cat(5), cat(4), f_ga, f_gb, nodal_forces,
            f_local, f_global, d_local, cat(16), cat(12), cat(15),
            cat(17), cat(18), cat(19), cat(20))


# speedup vs baseline: 1.0154x; 1.0154x over previous
import jax
import jax.numpy as jnp
from jax import lax
from jax.experimental import pallas as pl
from jax.experimental.pallas import tpu as pltpu
from jax.experimental.pallas import tpu_sc as plsc

NC = 2
NS = 16
NW = NC * NS
LANES = 16

C = 2000
VITERS = C // LANES

_MAGIC = 0x5F3759DF


def _rsqrt(x):
    xi = lax.bitcast_convert_type(x, jnp.int32)
    y = lax.bitcast_convert_type(
        jnp.int32(_MAGIC) - lax.shift_right_logical(xi, 1), jnp.float32)
    for _ in range(3):
        y = y * (1.5 - 0.5 * x * y * y)
    return y


def _beam_body(*refs):
    (tdx, tdy, tth, tcx, tcz, na1, nb1, p_e, p_a, p_i, z1) = refs[:11]
    outs_o = refs[11:32]
    (p0x_o, p0y_o, p0z_o, p1x_o, p1y_o, p1z_o) = refs[32:38]
    scr = refs[38:]
    ins = [list(scr[s * 15:(s + 1) * 15]) for s in range(2)]
    outs = list(scr[30:49])
    accx, accy, accz = scr[49:52]
    sem_lin, sem_gat, sem_out, sem_sc = scr[52:56]

    core = lax.axis_index("c")
    sub = lax.axis_index("s")
    wid = sub * NC + core
    n_edges = outs_o[0].shape[0]
    cpw = n_edges // (NW * C)
    assert cpw % 2 == 1 and cpw >= 3

    tables = (tdx, tdy, tth, tcx, tcz)

    def lin_pairs(ck, s):
        sl = pl.ds((wid * cpw + ck) * C, C)
        bufs = ins[s]
        return [(na1.at[sl], bufs[0]), (nb1.at[sl], bufs[1]),
                (p_e.at[sl], bufs[2]), (p_a.at[sl], bufs[3]),
                (p_i.at[sl], bufs[4])]

    def gat_pairs(s):
        bufs = ins[s]
        prs = [(tab.at[bufs[0]], bufs[5 + t]) for t, tab in enumerate(tables)]
        prs += [(tab.at[bufs[1]], bufs[10 + t]) for t, tab in enumerate(tables)]
        return prs

    def out_pairs(ck, s):
        sl = pl.ds((wid * cpw + ck) * C, C)
        prs = [(buf, ref.at[sl]) for buf, ref in zip(outs, (
            outs_o[0], outs_o[1], outs_o[2], outs_o[3], outs_o[4], outs_o[5],
            outs_o[6], outs_o[7], outs_o[8], outs_o[9],
            outs_o[10], outs_o[11], outs_o[13], outs_o[14],
            outs_o[16], outs_o[17], outs_o[18], outs_o[19], outs_o[20]))]
        prs += [(ins[s][7], outs_o[12].at[sl]),
                (ins[s][12], outs_o[15].at[sl])]
        return prs

    def sc_pairs(s):
        ia, ib = ins[s][0], ins[s][1]
        return [(outs[6], accx.at[ia]), (outs[7], accy.at[ia]),
                (outs[2], accz.at[ia]), (outs[8], accx.at[ib]),
                (outs[9], accy.at[ib]), (outs[5], accz.at[ib])]

    def issue(pairs, sem, add=False):
        for src, dst in pairs:
            pltpu.async_copy(src, dst, sem, add=add)

    def drain(pairs, sem):
        for src, dst in pairs:
            pltpu.make_async_copy(src, dst, sem).wait()

    def compute(s):
        bufs = ins[s]
        (f0_v, f1_v, f2_v, f3_v, f4_v, f5_v, g0_v, g1_v, g3_v, g4_v,
         ua_v, wa_v, ub_v, wb_v, ul_v, phi_v, l0_v, cc_v, ss_v) = outs

        def vec(i, _):
            sl = pl.ds(i * LANES, LANES)
            d_ax = bufs[5][sl]
            d_ay = bufs[6][sl]
            ta = bufs[7][sl]
            cax = bufs[8][sl]
            caz = bufs[9][sl]
            d_bx = bufs[10][sl]
            d_by = bufs[11][sl]
            tb = bufs[12][sl]
            cbx = bufs[13][sl]
            cbz = bufs[14][sl]
            pe = bufs[2][sl]
            pa = bufs[3][sl]
            pi = bufs[4][sl]

            dx0 = cbx - cax
            dz0 = cbz - caz
            x = dx0 * dx0 + dz0 * dz0
            rl = _rsqrt(x)
            l0 = x * rl
            cv = dx0 * rl
            sv = dz0 * rl
            ea = pe * pa
            ei = pe * pi
            ua = cv * d_ax + sv * d_ay
            wa = cv * d_ay - sv * d_ax
            ub = cv * d_bx + sv * d_by
            wb = cv * d_by - sv * d_bx
            rl2 = rl * rl
            rl3 = rl2 * rl
            f0 = ea * rl * (ua - ub)
            wab = wa - wb
            eil3 = ei * rl3
            eil2 = ei * rl2
            eil = ei * rl
            f1 = 12.0 * eil3 * wab + 6.0 * eil2 * (ta + tb)
            f2 = 6.0 * eil2 * wab + eil * (4.0 * ta + 2.0 * tb)
            f5 = 6.0 * eil2 * wab + eil * (2.0 * ta + 4.0 * tb)
            f3 = -f0
            f4 = -f1
            g0 = cv * f0 - sv * f1
            g1 = sv * f0 + cv * f1
            g3 = -g0
            g4 = -g1

            f0_v[sl] = f0
            f1_v[sl] = f1
            f2_v[sl] = f2
            f3_v[sl] = f3
            f4_v[sl] = f4
            f5_v[sl] = f5
            g0_v[sl] = g0
            g1_v[sl] = g1
            g3_v[sl] = g3
            g4_v[sl] = g4
            ua_v[sl] = ua
            wa_v[sl] = wa
            ub_v[sl] = ub
            wb_v[sl] = wb
            ul_v[sl] = ub - ua
            phi_v[sl] = (wb - wa) * rl
            l0_v[sl] = l0
            cc_v[sl] = cv
            ss_v[sl] = sv
            return 0

        lax.fori_loop(0, VITERS, vec, 0)

    @pl.when(sub == 0)
    def _zero():
        pltpu.sync_copy(z1, accx)
        pltpu.sync_copy(z1, accy)
        pltpu.sync_copy(z1, accz)

    plsc.subcore_barrier()

    issue(lin_pairs(0, 0), sem_lin)
    drain(lin_pairs(0, 0), sem_lin)
    issue(gat_pairs(0), sem_gat)
    issue(lin_pairs(1, 1), sem_lin)
    drain(gat_pairs(0), sem_gat)
    compute(0)
    issue(out_pairs(0, 0), sem_out)
    issue(sc_pairs(0), sem_sc, add=True)
    drain(lin_pairs(1, 1), sem_lin)
    issue(gat_pairs(1), sem_gat)

    def body(k, _):
        j1 = 2 * k + 1
        j2 = 2 * k + 2
        drain(out_pairs(j1 - 1, 0), sem_out)
        drain(sc_pairs(0), sem_sc)
        issue(lin_pairs(j2, 0), sem_lin)
        drain(gat_pairs(1), sem_gat)
        compute(1)
        issue(out_pairs(j1, 1), sem_out)
        issue(sc_pairs(1), sem_sc, add=True)
        drain(lin_pairs(j2, 0), sem_lin)
        issue(gat_pairs(0), sem_gat)
        drain(out_pairs(j1, 1), sem_out)
        drain(sc_pairs(1), sem_sc)

        @pl.when(k < (cpw - 3) // 2)
        def _pref():
            issue(lin_pairs(j2 + 1, 1), sem_lin)

        drain(gat_pairs(0), sem_gat)
        compute(0)
        issue(out_pairs(j2, 0), sem_out)
        issue(sc_pairs(0), sem_sc, add=True)

        @pl.when(k < (cpw - 3) // 2)
        def _gat():
            drain(lin_pairs(j2 + 1, 1), sem_lin)
            issue(gat_pairs(1), sem_gat)

        return 0

    lax.fori_loop(0, (cpw - 1) // 2, body, 0)
    drain(out_pairs(cpw - 1, 0), sem_out)
    drain(sc_pairs(0), sem_sc)

    plsc.subcore_barrier()

    @pl.when(jnp.logical_and(sub == 0, core == 0))
    def _out0():
        pltpu.sync_copy(accx, p0x_o)
        pltpu.sync_copy(accy, p0y_o)
        pltpu.sync_copy(accz, p0z_o)

    @pl.when(jnp.logical_and(sub == 0, core == 1))
    def _out1():
        pltpu.sync_copy(accx, p1x_o)
        pltpu.sync_copy(accy, p1y_o)
        pltpu.sync_copy(accz, p1z_o)


def kernel(pred_disp, coords, prop_E, prop_A, prop_I22, connectivity):
    n_nodes = pred_disp.shape[0]
    n_edges = connectivity.shape[0]
    assert n_edges % (NW * C) == 0

    f32 = jnp.float32
    tdx = pred_disp[:, 0].astype(f32)
    tdy = pred_disp[:, 1].astype(f32)
    tth = pred_disp[:, 2].astype(f32)
    tcx = coords[:, 0].astype(f32)
    tcz = coords[:, 2].astype(f32)
    na1 = connectivity[:, 0].astype(jnp.int32)
    nb1 = connectivity[:, 1].astype(jnp.int32)
    z1 = jnp.zeros((n_nodes,), f32)

    e1 = jax.ShapeDtypeStruct((n_edges,), f32)
    n1 = jax.ShapeDtypeStruct((n_nodes,), f32)
    out_type = (e1,) * 21 + (n1,) * 6

    set_bufs = ([pltpu.VMEM((C,), jnp.int32)] * 2 +
                [pltpu.VMEM((C,), f32)] * 3 +
                [pltpu.VMEM((C,), f32)] * 10)
    scratch = (
        set_bufs + set_bufs +
        [pltpu.VMEM((C,), f32)] * 19 +
        [pltpu.VMEM_SHARED((n_nodes,), f32)] * 3 +
        [pltpu.SemaphoreType.DMA] * 4
    )

    mesh = plsc.VectorSubcoreMesh(core_axis_name="c", subcore_axis_name="s",
                                  num_cores=NC, num_subcores=NS)
    run = pl.kernel(_beam_body, out_type=out_type, mesh=mesh,
                    scratch_types=scratch,
                    compiler_params=pltpu.CompilerParams(
                        needs_layout_passes=False))
    (f0, f1, f2, f3, f4, f5, g0, g1, g3, g4,
     ua, wa, ta, ub, wb, tb, u_l, phi, l0, cc, ss,
     p0x, p0y, p0z, p1x, p1y, p1z) = run(
         tdx, tdy, tth, tcx, tcz, na1, nb1,
         prop_E.astype(f32), prop_A.astype(f32), prop_I22.astype(f32), z1)

    f_local = jnp.stack([f0, f1, f2, f3, f4, f5], axis=1)
    f_global = jnp.stack([g0, g1, f2, g3, g4, f5], axis=1)
    d_local = jnp.stack([ua, wa, ta, ub, wb, tb], axis=1)
    f_ga = jnp.stack([g0, g1, f2], axis=1)
    f_gb = jnp.stack([g3, g4, f5], axis=1)
    nodal_forces = jnp.stack([p0x + p1x, p0y + p1y, p0z + p1z], axis=1)
    return (f3, f2, f5, f4, f_ga, f_gb, nodal_forces,
            f_local, f_global, d_local, u_l, ta, tb,
            phi, l0, cc, ss)
